# Initial kernel scaffold; baseline (speedup 1.0000x reference)
import functools

import jax
import jax.numpy as jnp
from jax import lax
from jax.experimental import pallas as pl
from jax.experimental.pallas import tpu as pltpu
from jax.experimental.pallas import tpu_sc as plsc

DM = 32
LANE = 128
K = 8


@functools.lru_cache(maxsize=None)
def _make_kernel(n_rows):
    info = plsc.get_sparse_core_info()
    nc, ns = info.num_cores, info.num_subcores
    nw = nc * ns
    rows_per_w = n_rows // nw
    n_chunks = rows_per_w // K
    mesh = plsc.VectorSubcoreMesh(core_axis_name="c", subcore_axis_name="s")

    @functools.partial(
        pl.kernel,
        mesh=mesh,
        out_type=jax.ShapeDtypeStruct((n_rows, LANE, DM), jnp.float32),
        scratch_types=[
            pltpu.VMEM((K, LANE), jnp.int32),
            pltpu.VMEM((K, LANE, DM), jnp.float32),
            pltpu.SemaphoreType.DMA,
        ],
    )
    def sc_gather(idx_hbm, table_hbm, out_hbm, idx_v, rows_v, sem):
        wid = lax.axis_index("s") * nc + lax.axis_index("c")
        r0 = wid * rows_per_w

        def chunk(i, carry):
            base = r0 + i * K
            pltpu.sync_copy(idx_hbm.at[pl.ds(base, K)], idx_v)
            copies = [
                pltpu.async_copy(table_hbm.at[idx_v.at[j]], rows_v.at[j], sem)
                for j in range(K)
            ]
            for c in copies:
                c.wait()
            pltpu.sync_copy(rows_v, out_hbm.at[pl.ds(base, K)])
            return carry

        lax.fori_loop(0, n_chunks, chunk, 0)

    return sc_gather


def kernel(x, W):
    b, s = x.shape
    n = b * s
    idx = x.reshape(n // LANE, LANE).astype(jnp.int32)
    out = _make_kernel(n // LANE)(idx, W)
    return out.reshape(b, s, DM)


# speedup vs baseline: 1.2848x; 1.2848x over previous
import functools

import jax
import jax.numpy as jnp
from jax import lax
from jax.experimental import pallas as pl
from jax.experimental.pallas import tpu as pltpu
from jax.experimental.pallas import tpu_sc as plsc

DM = 32
LANE = 128
K = 8


@functools.lru_cache(maxsize=None)
def _make_kernel(n_rows):
    info = plsc.get_sparse_core_info()
    nc, ns = info.num_cores, info.num_subcores
    nw = nc * ns
    rows_per_w = n_rows // nw
    n_chunks = rows_per_w // K
    mesh = plsc.VectorSubcoreMesh(core_axis_name="c", subcore_axis_name="s")

    @functools.partial(
        pl.kernel,
        mesh=mesh,
        compiler_params=pltpu.CompilerParams(use_tc_tiling_on_sc=False),
        out_type=jax.ShapeDtypeStruct((n_rows, LANE, DM), jnp.float32),
        scratch_types=[
            pltpu.VMEM((K, LANE), jnp.int32),
            pltpu.VMEM((K, LANE, DM), jnp.float32),
            pltpu.SemaphoreType.DMA,
        ],
    )
    def sc_gather(idx_hbm, table_hbm, out_hbm, idx_v, rows_v, sem):
        wid = lax.axis_index("s") * nc + lax.axis_index("c")
        r0 = wid * rows_per_w

        def chunk(i, carry):
            base = r0 + i * K
            pltpu.sync_copy(idx_hbm.at[pl.ds(base, K)], idx_v)
            copies = [
                pltpu.async_copy(table_hbm.at[idx_v.at[j]], rows_v.at[j], sem)
                for j in range(K)
            ]
            for c in copies:
                c.wait()
            pltpu.sync_copy(rows_v, out_hbm.at[pl.ds(base, K)])
            return carry

        lax.fori_loop(0, n_chunks, chunk, 0)

    return sc_gather


def kernel(x, W):
    b, s = x.shape
    n = b * s
    idx = x.reshape(n // LANE, LANE).astype(jnp.int32)
    out = _make_kernel(n // LANE)(idx, W)
    return out.reshape(b, s, DM)


# speedup vs baseline: 1.6803x; 1.3078x over previous
import functools

import jax
import jax.numpy as jnp
from jax import lax
from jax.experimental import pallas as pl
from jax.experimental.pallas import tpu as pltpu
from jax.experimental.pallas import tpu_sc as plsc

NB = 8


@functools.lru_cache(maxsize=None)
def _make_kernel(b, s, dm):
    info = plsc.get_sparse_core_info()
    nc, ns = info.num_cores, info.num_subcores
    nw = nc * ns
    b_per_w = b // nw
    n_chunks = b_per_w // NB
    mesh = plsc.VectorSubcoreMesh(core_axis_name="c", subcore_axis_name="s")

    @functools.partial(
        pl.kernel,
        mesh=mesh,
        compiler_params=pltpu.CompilerParams(use_tc_tiling_on_sc=False),
        out_type=jax.ShapeDtypeStruct((b, s, dm), jnp.float32),
        scratch_types=[
            pltpu.VMEM((NB, s), jnp.int32),
            pltpu.VMEM((NB, s, dm), jnp.float32),
            pltpu.SemaphoreType.DMA,
        ],
    )
    def sc_gather(idx_hbm, table_hbm, out_hbm, idx_v, rows_v, sem):
        wid = lax.axis_index("s") * nc + lax.axis_index("c")
        b0 = wid * b_per_w

        def chunk(i, carry):
            base = b0 + i * NB
            pltpu.sync_copy(idx_hbm.at[pl.ds(base, NB)], idx_v)
            copies = [
                pltpu.async_copy(table_hbm.at[idx_v.at[r]], rows_v.at[r], sem)
                for r in range(NB)
            ]
            for c in copies:
                c.wait()
            pltpu.sync_copy(rows_v, out_hbm.at[pl.ds(base, NB)])
            return carry

        lax.fori_loop(0, n_chunks, chunk, 0)

    return sc_gather


def kernel(x, W):
    b, s = x.shape
    return _make_kernel(b, s, W.shape[1])(x.astype(jnp.int32), W)
